# Initial kernel scaffold; baseline (speedup 1.0000x reference)
import functools

import jax
import jax.numpy as jnp
from jax import lax
from jax.experimental import pallas as pl
from jax.experimental.pallas import tpu as pltpu
from jax.experimental.pallas import tpu_sc as plsc

EPS = 1e-12
BN = 128
NTILE = 16
NPT = 1024 // NTILE
WIN = 1025


def _proj_body(h0, b0, h1, b1, side, inp, h2, b2, idx0, idx1, idx2, x0):
    s = side[...]
    pows = jnp.array([1, 2, 4, 8], jnp.int32)

    def ctx_idx(h_blk, b_blk):
        hm = h_blk.reshape(BN * 4, 4096)
        p = jax.lax.dot_general(
            hm, s.reshape(4096, 1),
            dimension_numbers=(((1,), (0,)), ((), ())),
            preferred_element_type=jnp.float32,
        ).reshape(BN, 4)
        bits = (p > b_blk).astype(jnp.int32)
        return jnp.sum(bits * pows[None, :], axis=1)

    idx0[...] = ctx_idx(h0[...], b0[...])
    idx1[...] = ctx_idx(h1[...], b1[...])

    @pl.when(pl.program_id(0) == 0)
    def _():
        p2 = jnp.sum(h2[...].reshape(4, 4096) * s[None, :], axis=1)
        bits2 = (p2 > b2[...].reshape(4)).astype(jnp.int32)
        i2 = jnp.sum(bits2 * pows)
        idx2[...] = jnp.full((8,), i2, jnp.int32)
        xc = jnp.clip(inp[...], EPS, 1.0 - EPS)
        x0[pl.ds(0, 1024)] = jnp.log(xc) - jnp.log1p(-xc)
        bs = jax.nn.sigmoid(jnp.float32(1.0))
        bl = jnp.log(bs) - jnp.log1p(-bs)
        x0[pl.ds(1024, 1)] = jnp.full((1,), bl, jnp.float32)


def _proj_call(inputs, side, H0, H1, H2, B0, B1, B2):
    grid = (1024 // BN,)
    return pl.pallas_call(
        _proj_body,
        grid=grid,
        in_specs=[
            pl.BlockSpec((BN, 4, 4096), lambda i: (i, 0, 0)),
            pl.BlockSpec((BN, 4), lambda i: (i, 0)),
            pl.BlockSpec((BN, 4, 4096), lambda i: (i, 0, 0)),
            pl.BlockSpec((BN, 4), lambda i: (i, 0)),
            pl.BlockSpec((4096,), lambda i: (0,)),
            pl.BlockSpec((1024,), lambda i: (0,)),
            pl.BlockSpec((1, 4, 4096), lambda i: (0, 0, 0)),
            pl.BlockSpec((1, 4), lambda i: (0, 0)),
        ],
        out_specs=[
            pl.BlockSpec((BN,), lambda i: (i,)),
            pl.BlockSpec((BN,), lambda i: (i,)),
            pl.BlockSpec((8,), lambda i: (0,)),
            pl.BlockSpec((WIN,), lambda i: (0,)),
        ],
        out_shape=[
            jax.ShapeDtypeStruct((1024,), jnp.int32),
            jax.ShapeDtypeStruct((1024,), jnp.int32),
            jax.ShapeDtypeStruct((8,), jnp.int32),
            jax.ShapeDtypeStruct((WIN,), jnp.float32),
        ],
    )(H0, B0, H1, B1, side, inputs, H2, B2)


def _row_dot(rows_ref, j, x_ref):
    zero = jnp.zeros((16,), jnp.float32)

    def body(c, accs):
        a0, a1, a2, a3 = accs
        b = c * 64
        a0 = a0 + rows_ref[j, pl.ds(b, 16)] * x_ref[pl.ds(b, 16)]
        a1 = a1 + rows_ref[j, pl.ds(b + 16, 16)] * x_ref[pl.ds(b + 16, 16)]
        a2 = a2 + rows_ref[j, pl.ds(b + 32, 16)] * x_ref[pl.ds(b + 32, 16)]
        a3 = a3 + rows_ref[j, pl.ds(b + 48, 16)] * x_ref[pl.ds(b + 48, 16)]
        return (a0, a1, a2, a3)

    a0, a1, a2, a3 = lax.fori_loop(0, 16, body, (zero, zero, zero, zero))
    return jnp.sum((a0 + a1) + (a2 + a3))


def _sc_body(w0, w1, w2, i0, i1, i2, x0h, out,
             idx_v, rowid_v, rows_v, x_v, x1_v, zbuf_v, sig_v,
             idx2_v, row2_v, zsh, sem):
    wid = lax.axis_index("s")
    base = wid * NPT
    iota = lax.iota(jnp.int32, 16)

    def gather_rows(w_hbm, idx_hbm):
        pltpu.sync_copy(idx_hbm.at[pl.ds(base, NPT)], idx_v)
        for j in range(NPT // 16):
            n = base + j * 16 + iota
            rowid_v[pl.ds(j * 16, 16)] = n * 16 + idx_v[pl.ds(j * 16, 16)]
        pltpu.async_copy(w_hbm.at[rowid_v], rows_v, sem).wait()

    def layer_dots(x_ref, x_tail):
        def per_neuron(j, carry):
            z = _row_dot(rows_v, j, x_ref)
            tail = rows_v[j, 1024] * x_tail(j)
            zbuf_v[j] = z + tail
            return carry
        lax.fori_loop(0, NPT, per_neuron, 0)

    def sigmoid_store(off):
        for j in range(NPT // 16):
            zc = zbuf_v[pl.ds(j * 16, 16)]
            sig_v[pl.ds(j * 16, 16)] = 1.0 / (1.0 + jnp.exp(-zc))
        pltpu.sync_copy(sig_v, out.at[pl.ds(off + base, NPT)])

    pltpu.sync_copy(x0h, x_v)
    gather_rows(w0, i0)
    layer_dots(x_v, lambda j: x_v[1024])
    sigmoid_store(0)
    pltpu.sync_copy(zbuf_v, zsh.at[pl.ds(base, NPT)])
    plsc.subcore_barrier()
    pltpu.sync_copy(zsh, x1_v)
    plsc.subcore_barrier()
    gather_rows(w1, i1)
    layer_dots(x1_v, lambda j: jnp.float32(1.0))
    sigmoid_store(1024)
    pltpu.sync_copy(zbuf_v, zsh.at[pl.ds(base, NPT)])
    plsc.subcore_barrier()

    @pl.when(wid == 0)
    def _():
        pltpu.sync_copy(zsh, x1_v)
        pltpu.sync_copy(i2, idx2_v)
        pltpu.async_copy(w2.at[idx2_v], row2_v, sem).wait()
        z2 = _row_dot(row2_v, 0, x1_v) + row2_v[0, 1024]
        sig_v[pl.ds(0, 16)] = 1.0 / (1.0 + jnp.exp(
            jnp.full((16,), -z2, jnp.float32)))
        pltpu.sync_copy(sig_v.at[pl.ds(0, 1)], out.at[pl.ds(2048, 1)])


def _sc_chain(W0f, W1f, W2f, idx0, idx1, idx2, x0):
    mesh = plsc.VectorSubcoreMesh(
        core_axis_name="c", subcore_axis_name="s", num_cores=1)
    f = functools.partial(
        pl.kernel,
        out_type=jax.ShapeDtypeStruct((2049,), jnp.float32),
        mesh=mesh,
        scratch_types=[
            pltpu.VMEM((NPT,), jnp.int32),
            pltpu.VMEM((NPT,), jnp.int32),
            pltpu.VMEM((NPT, WIN), jnp.float32),
            pltpu.VMEM((WIN,), jnp.float32),
            pltpu.VMEM((1024,), jnp.float32),
            pltpu.VMEM((NPT,), jnp.float32),
            pltpu.VMEM((NPT,), jnp.float32),
            pltpu.VMEM((8,), jnp.int32),
            pltpu.VMEM((8, WIN), jnp.float32),
            pltpu.VMEM_SHARED((1024,), jnp.float32),
            pltpu.SemaphoreType.DMA,
        ],
    )(_sc_body)
    return f(W0f, W1f, W2f, idx0, idx1, idx2, x0)


def kernel(inputs, side_info, W0, W1, W2, H0, H1, H2, B0, B1, B2):
    idx0, idx1, idx2, x0 = _proj_call(
        inputs, side_info, H0, H1, H2, B0, B1, B2)
    out = _sc_chain(
        W0.reshape(1024 * 16, WIN),
        W1.reshape(1024 * 16, WIN),
        W2.reshape(16, WIN),
        idx0, idx1, idx2, x0)
    return out


# speedup vs baseline: 1.3113x; 1.3113x over previous
import functools

import jax
import jax.numpy as jnp
from jax import lax
from jax.experimental import pallas as pl
from jax.experimental.pallas import tpu as pltpu
from jax.experimental.pallas import tpu_sc as plsc

EPS = 1e-12
BN = 128
NTILE = 16
NPT = 1024 // NTILE
WIN = 1025


def _proj_body(h0, b0, h1, b1, side, inp, h2, b2, w0b, w1b, w2b,
               idx0, idx1, idx2, x0, t0, t1, t2):
    s = side[...]

    def ctx_idx(h_blk, b_blk):
        hm = h_blk.reshape(BN * 4, 4096)
        p = jax.lax.dot_general(
            hm, s.reshape(4096, 1),
            dimension_numbers=(((1,), (0,)), ((), ())),
            preferred_element_type=jnp.float32,
        ).reshape(BN, 4)
        bits = (p > b_blk).astype(jnp.int32)
        pows = 1 << jax.lax.broadcasted_iota(jnp.int32, (BN, 4), 1)
        return jnp.sum(bits * pows, axis=1)

    def bias_tail(wb_blk, idx):
        plane = wb_blk[:, :, 0]
        cio = jax.lax.broadcasted_iota(jnp.int32, (BN, 16), 1)
        return jnp.sum(jnp.where(cio == idx[:, None], plane, 0.0), axis=1)

    i0v = ctx_idx(h0[...], b0[...])
    i1v = ctx_idx(h1[...], b1[...])
    idx0[...] = i0v
    idx1[...] = i1v

    bs = jax.nn.sigmoid(jnp.float32(1.0))
    bl = jnp.log(bs) - jnp.log1p(-bs)

    t0[...] = bias_tail(w0b[...], i0v) * bl
    t1[...] = bias_tail(w1b[...], i1v)

    @pl.when(pl.program_id(0) == 0)
    def _():
        p2 = jnp.sum(h2[...].reshape(1, 4, 4096) * s[None, None, :], axis=2)
        bits2 = (p2 > b2[...].reshape(1, 4)).astype(jnp.int32)
        pows2 = 1 << jax.lax.broadcasted_iota(jnp.int32, (1, 4), 1)
        i2 = jnp.sum(bits2 * pows2)
        idx2[...] = jnp.full((8,), i2, jnp.int32)
        cio2 = jax.lax.broadcasted_iota(jnp.int32, (1, 16), 1)
        t2sel = jnp.sum(jnp.where(cio2 == i2, w2b[:, :, 0], 0.0))
        t2[...] = jnp.full((16,), t2sel, jnp.float32)
        xc = jnp.clip(inp[...], EPS, 1.0 - EPS)
        x0[pl.ds(0, 1024)] = jnp.log(xc) - jnp.log1p(-xc)
        x0[pl.ds(1024, 1)] = jnp.full((1,), bl, jnp.float32)


def _proj_call(inputs, side, W0, W1, W2, H0, H1, H2, B0, B1, B2):
    grid = (1024 // BN,)
    return pl.pallas_call(
        _proj_body,
        grid=grid,
        in_specs=[
            pl.BlockSpec((BN, 4, 4096), lambda i: (i, 0, 0)),
            pl.BlockSpec((BN, 4), lambda i: (i, 0)),
            pl.BlockSpec((BN, 4, 4096), lambda i: (i, 0, 0)),
            pl.BlockSpec((BN, 4), lambda i: (i, 0)),
            pl.BlockSpec((4096,), lambda i: (0,)),
            pl.BlockSpec((1024,), lambda i: (0,)),
            pl.BlockSpec((1, 4, 4096), lambda i: (0, 0, 0)),
            pl.BlockSpec((1, 4), lambda i: (0, 0)),
            pl.BlockSpec((BN, 16, 128), lambda i: (i, 0, 8)),
            pl.BlockSpec((BN, 16, 128), lambda i: (i, 0, 8)),
            pl.BlockSpec((1, 16, 128), lambda i: (0, 0, 8)),
        ],
        out_specs=[
            pl.BlockSpec((BN,), lambda i: (i,)),
            pl.BlockSpec((BN,), lambda i: (i,)),
            pl.BlockSpec((8,), lambda i: (0,)),
            pl.BlockSpec((WIN,), lambda i: (0,)),
            pl.BlockSpec((BN,), lambda i: (i,)),
            pl.BlockSpec((BN,), lambda i: (i,)),
            pl.BlockSpec((16,), lambda i: (0,)),
        ],
        out_shape=[
            jax.ShapeDtypeStruct((1024,), jnp.int32),
            jax.ShapeDtypeStruct((1024,), jnp.int32),
            jax.ShapeDtypeStruct((8,), jnp.int32),
            jax.ShapeDtypeStruct((WIN,), jnp.float32),
            jax.ShapeDtypeStruct((1024,), jnp.float32),
            jax.ShapeDtypeStruct((1024,), jnp.float32),
            jax.ShapeDtypeStruct((16,), jnp.float32),
        ],
    )(H0, B0, H1, B1, side, inputs, H2, B2, W0, W1, W2)


def _sc_body(w0, w1, w2, i0, i1, i2, x0h, t0h, t1h, t2h, out,
             idx_v, rowid_v, rows_v, x_v, x1_v, tail_v, zbuf_v, sig_v,
             idx2_v, row2_v, t2_v, acc1d_v, zsh, sem):
    wid = lax.axis_index("s")
    base = wid * NPT
    iota = lax.iota(jnp.int32, 16)

    def gather_rows(w_hbm, idx_hbm):
        pltpu.sync_copy(idx_hbm.at[pl.ds(base, NPT)], idx_v)
        for j in range(NPT // 16):
            n = base + j * 16 + iota
            rowid_v[pl.ds(j * 16, 16)] = n * 16 + idx_v[pl.ds(j * 16, 16)]
        pltpu.async_copy(
            w_hbm.at[rowid_v, pl.ds(0, 1024)], rows_v, sem).wait()

    def layer_dots(x_ref):
        for g in range(NPT // 16):
            r0 = g * 16

            def body(c, accs):
                xc = x_ref[pl.ds(c * 16, 16)]
                return tuple(
                    accs[k] + rows_v[r0 + k, pl.ds(c * 16, 16)] * xc
                    for k in range(16))

            zero = jnp.zeros((16,), jnp.float32)
            accs = lax.fori_loop(0, 64, body, (zero,) * 16)
            for k in range(16):
                acc1d_v[pl.ds(k * 16, 16)] = accs[k]
            z = tail_v[pl.ds(r0, 16)]
            for c in range(16):
                z = z + plsc.load_gather(acc1d_v, [iota * 16 + c])
            zbuf_v[pl.ds(r0, 16)] = z
            sig_v[pl.ds(r0, 16)] = 1.0 / (1.0 + jnp.exp(-z))

    pltpu.sync_copy(x0h, x_v)
    pltpu.sync_copy(t0h.at[pl.ds(base, NPT)], tail_v)
    gather_rows(w0, i0)
    layer_dots(x_v)
    pltpu.sync_copy(sig_v, out.at[pl.ds(base, NPT)])
    pltpu.sync_copy(zbuf_v, zsh.at[pl.ds(base, NPT)])
    plsc.subcore_barrier()
    pltpu.sync_copy(zsh, x1_v)
    plsc.subcore_barrier()
    pltpu.sync_copy(t1h.at[pl.ds(base, NPT)], tail_v)
    gather_rows(w1, i1)
    layer_dots(x1_v)
    pltpu.sync_copy(sig_v, out.at[pl.ds(1024 + base, NPT)])
    pltpu.sync_copy(zbuf_v, zsh.at[pl.ds(base, NPT)])
    plsc.subcore_barrier()

    @pl.when(wid == 0)
    def _():
        pltpu.sync_copy(zsh, x1_v)
        pltpu.sync_copy(i2, idx2_v)
        pltpu.sync_copy(t2h, t2_v)
        pltpu.async_copy(
            w2.at[idx2_v, pl.ds(0, 1024)], row2_v, sem).wait()

        def body(c, acc):
            return acc + row2_v[0, pl.ds(c * 16, 16)] * x1_v[pl.ds(c * 16, 16)]

        acc = lax.fori_loop(0, 64, body, jnp.zeros((16,), jnp.float32))
        t2vec = t2_v[pl.ds(0, 16)]
        acc = acc + jnp.where(iota == 0, t2vec, 0.0)
        acc1d_v[pl.ds(0, 16)] = acc
        z2 = jnp.zeros((16,), jnp.float32)
        for c in range(16):
            z2 = z2 + plsc.load_gather(acc1d_v, [iota * 0 + c])
        sig_v[pl.ds(0, 16)] = 1.0 / (1.0 + jnp.exp(-z2))
        pltpu.sync_copy(sig_v.at[pl.ds(0, 1)], out.at[pl.ds(2048, 1)])


def _sc_chain(W0f, W1f, W2f, idx0, idx1, idx2, x0, t0, t1, t2):
    mesh = plsc.VectorSubcoreMesh(
        core_axis_name="c", subcore_axis_name="s", num_cores=1)
    f = functools.partial(
        pl.kernel,
        out_type=jax.ShapeDtypeStruct((2049,), jnp.float32),
        mesh=mesh,
        compiler_params=pltpu.CompilerParams(needs_layout_passes=False),
        scratch_types=[
            pltpu.VMEM((NPT,), jnp.int32),
            pltpu.VMEM((NPT,), jnp.int32),
            pltpu.VMEM((NPT, 1024), jnp.float32),
            pltpu.VMEM((WIN,), jnp.float32),
            pltpu.VMEM((1024,), jnp.float32),
            pltpu.VMEM((NPT,), jnp.float32),
            pltpu.VMEM((NPT,), jnp.float32),
            pltpu.VMEM((NPT,), jnp.float32),
            pltpu.VMEM((8,), jnp.int32),
            pltpu.VMEM((8, 1024), jnp.float32),
            pltpu.VMEM((16,), jnp.float32),
            pltpu.VMEM((256,), jnp.float32),
            pltpu.VMEM_SHARED((1024,), jnp.float32),
            pltpu.SemaphoreType.DMA,
        ],
    )(_sc_body)
    return f(W0f, W1f, W2f, idx0, idx1, idx2, x0, t0, t1, t2)


def kernel(inputs, side_info, W0, W1, W2, H0, H1, H2, B0, B1, B2):
    idx0, idx1, idx2, x0, t0, t1, t2 = _proj_call(
        inputs, side_info, W0, W1, W2, H0, H1, H2, B0, B1, B2)
    out = _sc_chain(
        W0.reshape(1024 * 16, WIN),
        W1.reshape(1024 * 16, WIN),
        W2.reshape(16, WIN),
        idx0, idx1, idx2, x0, t0, t1, t2)
    return out


# speedup vs baseline: 1.6680x; 1.2721x over previous
import functools
import math

import jax
import jax.numpy as jnp
import numpy as np
from jax import lax
from jax.experimental import pallas as pl
from jax.experimental.pallas import tpu as pltpu
from jax.experimental.pallas import tpu_sc as plsc

EPS = 1e-12
BN = 128
NTILE = 16
NPT = 1024 // NTILE
WIN = 1025

_BS = 1.0 / (1.0 + math.exp(-1.0))
BIAS_LOGIT = float(np.float32(math.log(_BS) - math.log1p(-_BS)))


def _proj_body(h0, b0, h1, b1, side, inp, h2, b2, idx0, idx1, idx2, x0):
    s = side[...]

    def ctx_idx(h_blk, b_blk):
        hm = h_blk.reshape(BN * 4, 4096)
        p = jax.lax.dot_general(
            hm, s.reshape(4096, 1),
            dimension_numbers=(((1,), (0,)), ((), ())),
            preferred_element_type=jnp.float32,
        ).reshape(BN, 4)
        bits = (p > b_blk).astype(jnp.int32)
        pows = 1 << jax.lax.broadcasted_iota(jnp.int32, (BN, 4), 1)
        return jnp.sum(bits * pows, axis=1)

    idx0[...] = ctx_idx(h0[...], b0[...])
    idx1[...] = ctx_idx(h1[...], b1[...])

    @pl.when(pl.program_id(0) == 0)
    def _():
        p2 = jnp.sum(h2[...].reshape(1, 4, 4096) * s[None, None, :], axis=2)
        bits2 = (p2 > b2[...].reshape(1, 4)).astype(jnp.int32)
        pows2 = 1 << jax.lax.broadcasted_iota(jnp.int32, (1, 4), 1)
        i2 = jnp.sum(bits2 * pows2)
        idx2[...] = jnp.full((16,), i2, jnp.int32)
        xc = jnp.clip(inp[...], EPS, 1.0 - EPS)
        x0[pl.ds(0, 1024)] = jnp.log(xc) - jnp.log1p(-xc)
        x0[pl.ds(1024, 1)] = jnp.full((1,), BIAS_LOGIT, jnp.float32)


def _proj_call(inputs, side, H0, H1, H2, B0, B1, B2):
    grid = (1024 // BN,)
    return pl.pallas_call(
        _proj_body,
        grid=grid,
        in_specs=[
            pl.BlockSpec((BN, 4, 4096), lambda i: (i, 0, 0)),
            pl.BlockSpec((BN, 4), lambda i: (i, 0)),
            pl.BlockSpec((BN, 4, 4096), lambda i: (i, 0, 0)),
            pl.BlockSpec((BN, 4), lambda i: (i, 0)),
            pl.BlockSpec((4096,), lambda i: (0,)),
            pl.BlockSpec((1024,), lambda i: (0,)),
            pl.BlockSpec((1, 4, 4096), lambda i: (0, 0, 0)),
            pl.BlockSpec((1, 4), lambda i: (0, 0)),
        ],
        out_specs=[
            pl.BlockSpec((BN,), lambda i: (i,)),
            pl.BlockSpec((BN,), lambda i: (i,)),
            pl.BlockSpec((16,), lambda i: (0,)),
            pl.BlockSpec((WIN,), lambda i: (0,)),
        ],
        out_shape=[
            jax.ShapeDtypeStruct((1024,), jnp.int32),
            jax.ShapeDtypeStruct((1024,), jnp.int32),
            jax.ShapeDtypeStruct((16,), jnp.int32),
            jax.ShapeDtypeStruct((WIN,), jnp.float32),
        ],
    )(H0, B0, H1, B1, side, inputs, H2, B2)


def _sc_body(w0, w1, w2, i0, i1, i2, x0h, wb0, wb1, wb2, out,
             idx_v, rowid_v, lrow_v, rows_v, x_v, x1_v, wb_v, tail_v,
             zbuf_v, sig_v, idx2_v, row2_v, wb2_v, acc1d_v, zsh, sem):
    wid = lax.axis_index("s")
    base = wid * NPT
    iota = lax.iota(jnp.int32, 16)

    def gather_rows(w_hbm, wb_hbm, idx_hbm, tail_scale):
        pltpu.sync_copy(idx_hbm.at[pl.ds(base, NPT)], idx_v)
        pltpu.sync_copy(wb_hbm.at[pl.ds(base * 16, NPT * 16)], wb_v)
        for j in range(NPT // 16):
            lrow = (j * 16 + iota) * 16 + idx_v[pl.ds(j * 16, 16)]
            lrow_v[pl.ds(j * 16, 16)] = lrow
            rowid_v[pl.ds(j * 16, 16)] = base * 16 + lrow
        pltpu.async_copy(
            w_hbm.at[rowid_v, pl.ds(0, 1024)], rows_v, sem).wait()
        for j in range(NPT // 16):
            t = plsc.load_gather(wb_v, [lrow_v[pl.ds(j * 16, 16)]])
            tail_v[pl.ds(j * 16, 16)] = t * tail_scale

    def layer_dots(x_ref):
        for g in range(NPT // 16):
            r0 = g * 16

            def body(c, accs):
                xc = x_ref[pl.ds(c * 16, 16)]
                return tuple(
                    accs[k] + rows_v[r0 + k, pl.ds(c * 16, 16)] * xc
                    for k in range(16))

            zero = jnp.zeros((16,), jnp.float32)
            accs = lax.fori_loop(0, 64, body, (zero,) * 16)
            for k in range(16):
                acc1d_v[pl.ds(k * 16, 16)] = accs[k]
            z = tail_v[pl.ds(r0, 16)]
            for c in range(16):
                z = z + plsc.load_gather(acc1d_v, [iota * 16 + c])
            zbuf_v[pl.ds(r0, 16)] = z
            sig_v[pl.ds(r0, 16)] = 1.0 / (1.0 + jnp.exp(-z))

    pltpu.sync_copy(x0h, x_v)
    gather_rows(w0, wb0, i0, jnp.float32(BIAS_LOGIT))
    layer_dots(x_v)
    pltpu.sync_copy(sig_v, out.at[pl.ds(base, NPT)])
    pltpu.sync_copy(zbuf_v, zsh.at[pl.ds(base, NPT)])
    plsc.subcore_barrier()
    pltpu.sync_copy(zsh, x1_v)
    plsc.subcore_barrier()
    gather_rows(w1, wb1, i1, jnp.float32(1.0))
    layer_dots(x1_v)
    pltpu.sync_copy(sig_v, out.at[pl.ds(1024 + base, NPT)])
    pltpu.sync_copy(zbuf_v, zsh.at[pl.ds(base, NPT)])
    plsc.subcore_barrier()

    @pl.when(wid == 0)
    def _():
        pltpu.sync_copy(zsh, x1_v)
        pltpu.sync_copy(i2, idx2_v)
        pltpu.sync_copy(wb2, wb2_v)
        pltpu.async_copy(
            w2.at[idx2_v, pl.ds(0, 1024)], row2_v, sem).wait()

        def body(c, acc):
            return acc + row2_v[0, pl.ds(c * 16, 16)] * x1_v[pl.ds(c * 16, 16)]

        acc = lax.fori_loop(0, 64, body, jnp.zeros((16,), jnp.float32))
        t2vec = plsc.load_gather(wb2_v, [idx2_v[pl.ds(0, 16)]])
        acc = acc + jnp.where(iota == 0, t2vec, 0.0)
        acc1d_v[pl.ds(0, 16)] = acc
        z2 = jnp.zeros((16,), jnp.float32)
        for c in range(16):
            z2 = z2 + plsc.load_gather(acc1d_v, [iota * 0 + c])
        sig_v[pl.ds(0, 16)] = 1.0 / (1.0 + jnp.exp(-z2))
        pltpu.sync_copy(sig_v.at[pl.ds(0, 1)], out.at[pl.ds(2048, 1)])


def _sc_chain(W0f, W1f, W2f, idx0, idx1, idx2, x0, wb0, wb1, wb2):
    mesh = plsc.VectorSubcoreMesh(
        core_axis_name="c", subcore_axis_name="s", num_cores=1)
    f = functools.partial(
        pl.kernel,
        out_type=jax.ShapeDtypeStruct((2049,), jnp.float32),
        mesh=mesh,
        compiler_params=pltpu.CompilerParams(needs_layout_passes=False),
        scratch_types=[
            pltpu.VMEM((NPT,), jnp.int32),
            pltpu.VMEM((NPT,), jnp.int32),
            pltpu.VMEM((NPT,), jnp.int32),
            pltpu.VMEM((NPT, 1024), jnp.float32),
            pltpu.VMEM((WIN,), jnp.float32),
            pltpu.VMEM((1024,), jnp.float32),
            pltpu.VMEM((NPT * 16,), jnp.float32),
            pltpu.VMEM((NPT,), jnp.float32),
            pltpu.VMEM((NPT,), jnp.float32),
            pltpu.VMEM((NPT,), jnp.float32),
            pltpu.VMEM((16,), jnp.int32),
            pltpu.VMEM((16, 1024), jnp.float32),
            pltpu.VMEM((16,), jnp.float32),
            pltpu.VMEM((256,), jnp.float32),
            pltpu.VMEM_SHARED((1024,), jnp.float32),
            pltpu.SemaphoreType.DMA,
        ],
    )(_sc_body)
    return f(W0f, W1f, W2f, idx0, idx1, idx2, x0, wb0, wb1, wb2)


def kernel(inputs, side_info, W0, W1, W2, H0, H1, H2, B0, B1, B2):
    idx0, idx1, idx2, x0 = _proj_call(
        inputs, side_info, H0, H1, H2, B0, B1, B2)
    wb0 = W0[:, :, 1024].reshape(1024 * 16)
    wb1 = W1[:, :, 1024].reshape(1024 * 16)
    wb2 = W2[:, :, 1024].reshape(16)
    out = _sc_chain(
        W0.reshape(1024 * 16, WIN),
        W1.reshape(1024 * 16, WIN),
        W2.reshape(16, WIN),
        idx0, idx1, idx2, x0, wb0, wb1, wb2)
    return out


# speedup vs baseline: 1.6739x; 1.0035x over previous
import functools
import math

import jax
import jax.numpy as jnp
import numpy as np
from jax import lax
from jax.experimental import pallas as pl
from jax.experimental.pallas import tpu as pltpu
from jax.experimental.pallas import tpu_sc as plsc

EPS = 1e-12
BN = 128
NTILE = 16
NPT = 1024 // NTILE
WIN = 1025

_BS = 1.0 / (1.0 + math.exp(-1.0))
BIAS_LOGIT = float(np.float32(math.log(_BS) - math.log1p(-_BS)))


def _proj_body(h0, b0, h1, b1, side, inp, h2, b2, idx0, idx1, idx2, x0):
    s = side[...]

    def ctx_idx(h_blk, b_blk):
        hm = h_blk.reshape(BN * 4, 4096)
        p = jax.lax.dot_general(
            hm, s.reshape(4096, 1),
            dimension_numbers=(((1,), (0,)), ((), ())),
            preferred_element_type=jnp.float32,
        ).reshape(BN, 4)
        bits = (p > b_blk).astype(jnp.int32)
        pows = 1 << jax.lax.broadcasted_iota(jnp.int32, (BN, 4), 1)
        return jnp.sum(bits * pows, axis=1)

    idx0[...] = ctx_idx(h0[...], b0[...])
    idx1[...] = ctx_idx(h1[...], b1[...])

    @pl.when(pl.program_id(0) == 0)
    def _():
        p2 = jnp.sum(h2[...].reshape(1, 4, 4096) * s[None, None, :], axis=2)
        bits2 = (p2 > b2[...].reshape(1, 4)).astype(jnp.int32)
        pows2 = 1 << jax.lax.broadcasted_iota(jnp.int32, (1, 4), 1)
        i2 = jnp.sum(bits2 * pows2)
        idx2[...] = jnp.full((16,), i2, jnp.int32)
        xc = jnp.clip(inp[...], EPS, 1.0 - EPS)
        x0[pl.ds(0, 1024)] = jnp.log(xc) - jnp.log1p(-xc)
        x0[pl.ds(1024, 1)] = jnp.full((1,), BIAS_LOGIT, jnp.float32)


def _proj_call(inputs, side, H0, H1, H2, B0, B1, B2):
    grid = (1024 // BN,)
    return pl.pallas_call(
        _proj_body,
        grid=grid,
        in_specs=[
            pl.BlockSpec((BN, 4, 4096), lambda i: (i, 0, 0)),
            pl.BlockSpec((BN, 4), lambda i: (i, 0)),
            pl.BlockSpec((BN, 4, 4096), lambda i: (i, 0, 0)),
            pl.BlockSpec((BN, 4), lambda i: (i, 0)),
            pl.BlockSpec((4096,), lambda i: (0,)),
            pl.BlockSpec((1024,), lambda i: (0,)),
            pl.BlockSpec((1, 4, 4096), lambda i: (0, 0, 0)),
            pl.BlockSpec((1, 4), lambda i: (0, 0)),
        ],
        out_specs=[
            pl.BlockSpec((BN,), lambda i: (i,)),
            pl.BlockSpec((BN,), lambda i: (i,)),
            pl.BlockSpec((16,), lambda i: (0,)),
            pl.BlockSpec((WIN,), lambda i: (0,)),
        ],
        out_shape=[
            jax.ShapeDtypeStruct((1024,), jnp.int32),
            jax.ShapeDtypeStruct((1024,), jnp.int32),
            jax.ShapeDtypeStruct((16,), jnp.int32),
            jax.ShapeDtypeStruct((WIN,), jnp.float32),
        ],
    )(H0, B0, H1, B1, side, inputs, H2, B2)


def _sc_body(w0, w1, w2, i0, i1, i2, x0h, out,
             idx_v, rowid_v, rows_v, x_v, x1_v,
             zbuf_v, sig_v, idx2_v, row2_v, acc1d_v, zsh, sem):
    wid = lax.axis_index("s")
    base = wid * NPT
    iota = lax.iota(jnp.int32, 16)

    def gather_rows(w_hbm, idx_hbm):
        pltpu.sync_copy(idx_hbm.at[pl.ds(base, NPT)], idx_v)
        for j in range(NPT // 16):
            n = base + j * 16 + iota
            rowid_v[pl.ds(j * 16, 16)] = n * 16 + idx_v[pl.ds(j * 16, 16)]
        pltpu.async_copy(
            w_hbm.at[rowid_v, pl.ds(0, 1152)], rows_v, sem).wait()

    lane0 = iota == 0

    def layer_dots(x_ref, tail_scale):
        for g in range(NPT // 16):
            r0 = g * 16

            def body(c, accs):
                xc = x_ref[pl.ds(c * 16, 16)]
                return tuple(
                    accs[k] + rows_v[r0 + k, pl.ds(c * 16, 16)] * xc
                    for k in range(16))

            zero = jnp.zeros((16,), jnp.float32)
            accs = lax.fori_loop(0, 64, body, (zero,) * 16)
            for k in range(16):
                tvec = rows_v[r0 + k, pl.ds(1024, 16)]
                a = accs[k] + jnp.where(lane0, tvec * tail_scale, 0.0)
                acc1d_v[pl.ds(k * 16, 16)] = a
            z = jnp.zeros((16,), jnp.float32)
            for c in range(16):
                z = z + plsc.load_gather(acc1d_v, [iota * 16 + c])
            zbuf_v[pl.ds(r0, 16)] = z
            sig_v[pl.ds(r0, 16)] = 1.0 / (1.0 + jnp.exp(-z))

    pltpu.sync_copy(x0h, x_v)
    gather_rows(w0, i0)
    layer_dots(x_v, jnp.float32(BIAS_LOGIT))
    pltpu.sync_copy(sig_v, out.at[pl.ds(base, NPT)])
    pltpu.sync_copy(zbuf_v, zsh.at[pl.ds(base, NPT)])
    plsc.subcore_barrier()
    pltpu.sync_copy(zsh, x1_v)
    plsc.subcore_barrier()
    gather_rows(w1, i1)
    layer_dots(x1_v, jnp.float32(1.0))
    pltpu.sync_copy(sig_v, out.at[pl.ds(1024 + base, NPT)])
    pltpu.sync_copy(zbuf_v, zsh.at[pl.ds(base, NPT)])
    plsc.subcore_barrier()

    @pl.when(wid == 0)
    def _():
        pltpu.sync_copy(zsh, x1_v)
        pltpu.sync_copy(i2, idx2_v)
        pltpu.async_copy(
            w2.at[idx2_v, pl.ds(0, 1152)], row2_v, sem).wait()

        def body(c, acc):
            return acc + row2_v[0, pl.ds(c * 16, 16)] * x1_v[pl.ds(c * 16, 16)]

        acc = lax.fori_loop(0, 64, body, jnp.zeros((16,), jnp.float32))
        t2vec = row2_v[0, pl.ds(1024, 16)]
        acc = acc + jnp.where(lane0, t2vec, 0.0)
        acc1d_v[pl.ds(0, 16)] = acc
        z2 = jnp.zeros((16,), jnp.float32)
        for c in range(16):
            z2 = z2 + plsc.load_gather(acc1d_v, [iota * 0 + c])
        sig_v[pl.ds(0, 16)] = 1.0 / (1.0 + jnp.exp(-z2))
        pltpu.sync_copy(sig_v.at[pl.ds(0, 1)], out.at[pl.ds(2048, 1)])


def _sc_chain(W0f, W1f, W2f, idx0, idx1, idx2, x0):
    mesh = plsc.VectorSubcoreMesh(
        core_axis_name="c", subcore_axis_name="s", num_cores=1)
    f = functools.partial(
        pl.kernel,
        out_type=jax.ShapeDtypeStruct((2049,), jnp.float32),
        mesh=mesh,
        compiler_params=pltpu.CompilerParams(needs_layout_passes=False),
        scratch_types=[
            pltpu.VMEM((NPT,), jnp.int32),
            pltpu.VMEM((NPT,), jnp.int32),
            pltpu.VMEM((NPT, 1152), jnp.float32),
            pltpu.VMEM((WIN,), jnp.float32),
            pltpu.VMEM((1024,), jnp.float32),
            pltpu.VMEM((NPT,), jnp.float32),
            pltpu.VMEM((NPT,), jnp.float32),
            pltpu.VMEM((16,), jnp.int32),
            pltpu.VMEM((16, 1152), jnp.float32),
            pltpu.VMEM((256,), jnp.float32),
            pltpu.VMEM_SHARED((1024,), jnp.float32),
            pltpu.SemaphoreType.DMA,
        ],
    )(_sc_body)
    return f(W0f, W1f, W2f, idx0, idx1, idx2, x0)


def kernel(inputs, side_info, W0, W1, W2, H0, H1, H2, B0, B1, B2):
    idx0, idx1, idx2, x0 = _proj_call(
        inputs, side_info, H0, H1, H2, B0, B1, B2)
    out = _sc_chain(
        W0.reshape(1024 * 16, WIN),
        W1.reshape(1024 * 16, WIN),
        W2.reshape(16, WIN),
        idx0, idx1, idx2, x0)
    return out
